# Initial kernel scaffold; baseline (speedup 1.0000x reference)
import jax
import jax.numpy as jnp
from jax.experimental import pallas as pl


def kernel(features, feat_memory, pred_memory):
    raise NotImplementedError("write your pallas kernel here")


# speedup vs baseline: 58.0294x; 58.0294x over previous
import functools

import jax
import jax.numpy as jnp
from jax import lax
from jax.experimental import pallas as pl
from jax.experimental.pallas import tpu as pltpu

_K = 5
_TOPK = _K + 1
_RUN = 128
_BIG_IDX = 2**30


def _knn_body(n_valid, num_tiles, tn, f_ref, m_ref, idx_ref, vals, idxs):
    i = pl.program_id(0)
    b = f_ref.shape[0]

    @pl.when(i == 0)
    def _init():
        vals[...] = jnp.full(vals.shape, jnp.inf, jnp.float32)
        idxs[...] = jnp.full(idxs.shape, _BIG_IDX, jnp.int32)

    f = f_ref[...]
    norm = jnp.sqrt(jnp.sum(f * f, axis=1, keepdims=True))
    fn = f / jnp.clip(norm, 1e-12, None)
    q_sq = jnp.sum(fn * fn, axis=1, keepdims=True)

    m = m_ref[...]
    ones_row = jnp.ones((1, m.shape[1]), jnp.float32)
    m_sq = lax.dot_general(ones_row, m * m, (((1,), (1,)), ((), ())),
                           preferred_element_type=jnp.float32)
    s = lax.dot_general(fn, m, (((1,), (1,)), ((), ())),
                        precision=lax.Precision.HIGHEST,
                        preferred_element_type=jnp.float32)
    dist = (q_sq - 2.0 * s) + m_sq

    gcol = i * tn + lax.broadcasted_iota(jnp.int32, (1, tn), 1)
    dist = jnp.where(gcol < n_valid, dist, jnp.inf)

    cur_v = jnp.concatenate([vals[...], dist], axis=1)
    cur_i = jnp.concatenate(
        [idxs[...], jnp.broadcast_to(gcol, (b, tn))], axis=1)

    new_v, new_i = [], []
    for _ in range(_TOPK):
        v = jnp.min(cur_v, axis=1, keepdims=True)
        pick = jnp.min(jnp.where(cur_v == v, cur_i, _BIG_IDX),
                       axis=1, keepdims=True)
        new_v.append(v)
        new_i.append(pick)
        cur_v = jnp.where(cur_i == pick, jnp.inf, cur_v)
    vals[:, 0:_TOPK] = jnp.concatenate(new_v, axis=1)
    idxs[:, 0:_TOPK] = jnp.concatenate(new_i, axis=1)

    @pl.when(i == num_tiles - 1)
    def _emit():
        idx_ref[...] = idxs[:, 0:8]


def _topk_indices(features, feat_memory, tn=2048):
    b, d = features.shape
    n = feat_memory.shape[0]
    num_tiles = (n + tn - 1) // tn
    n_pad = num_tiles * tn
    if n_pad != n:
        feat_memory = jnp.pad(feat_memory, ((0, n_pad - n), (0, 0)))
    grid = (num_tiles,)
    return pl.pallas_call(
        functools.partial(_knn_body, n, num_tiles, tn),
        grid=grid,
        in_specs=[
            pl.BlockSpec((b, d), lambda i: (0, 0)),
            pl.BlockSpec((tn, d), lambda i: (i, 0)),
        ],
        out_specs=pl.BlockSpec((b, 8), lambda i: (0, 0)),
        out_shape=jax.ShapeDtypeStruct((b, 8), jnp.int32),
        scratch_shapes=[
            pltpu.VMEM((b, _RUN), jnp.float32),
            pltpu.VMEM((b, _RUN), jnp.int32),
        ],
    )(features, feat_memory)


def kernel(features, feat_memory, pred_memory):
    idx8 = _topk_indices(features, feat_memory)
    neigh = idx8[:, 1:1 + _K]
    logits = jnp.mean(jnp.take(pred_memory, neigh, axis=0), axis=1)
    pseudo_labels = jnp.argmax(logits, axis=1)
    return (pseudo_labels, logits)
